# baseline (speedup 1.0000x reference)
import functools

import jax
import jax.numpy as jnp
from jax.experimental import pallas as pl

_ROW = 1000
_PAD = 1024
_BLK = 128


def _sort_block(x_ref, o_ref):
    x = x_ref[...]
    neg = jnp.full((_BLK, _PAD - _ROW), -jnp.inf, dtype=jnp.float32)
    v = jnp.concatenate([x, neg], axis=1).T
    idx = jax.lax.broadcasted_iota(jnp.int32, (_PAD, _BLK), 0)
    row = jax.lax.broadcasted_iota(jnp.int32, (_PAD, _BLK), 0)

    k = 2
    while k <= _PAD:
        j = k // 2
        while j >= 1:
            g = _PAD // (2 * j)
            v4 = v.reshape(g, 2, j, _BLK)
            i4 = idx.reshape(g, 2, j, _BLK)
            d4 = ((row & k) != 0).reshape(g, 2, j, _BLK)[:, 0]
            va, vb = v4[:, 0], v4[:, 1]
            ia, ib = i4[:, 0], i4[:, 1]
            lt = (vb > va) | ((vb == va) & (ib < ia))
            swap = lt != d4
            na_v = jnp.where(swap, vb, va)
            nb_v = jnp.where(swap, va, vb)
            na_i = jnp.where(swap, ib, ia)
            nb_i = jnp.where(swap, ia, ib)
            v = jnp.stack([na_v, nb_v], axis=1).reshape(_PAD, _BLK)
            idx = jnp.stack([na_i, nb_i], axis=1).reshape(_PAD, _BLK)
            j //= 2
        k *= 2

    o_ref[...] = idx.T[:, :_ROW]


@jax.jit
def kernel(inputs):
    n_rows = inputs.size // _ROW
    x = inputs.reshape(n_rows, _ROW)
    grid = n_rows // _BLK
    return pl.pallas_call(
        _sort_block,
        grid=(grid,),
        in_specs=[pl.BlockSpec((_BLK, _ROW), lambda i: (i, 0))],
        out_specs=pl.BlockSpec((_BLK, _ROW), lambda i: (i, 0)),
        out_shape=jax.ShapeDtypeStruct((n_rows, _ROW), jnp.int32),
    )(x)


# speedup vs baseline: 3.6436x; 3.6436x over previous
import functools

import jax
import jax.numpy as jnp
from jax import lax
from jax.experimental import pallas as pl
from jax.experimental.pallas import tpu as pltpu
from jax.experimental.pallas import tpu_sc as plsc

ROW = 1000
PPAD = 1008
PITCH = 17
GROUP = 16
NW = 32
TW = PPAD * PITCH
INT_MIN = jnp.int32(-2147483648)


def _make(n_rows):
    rows_per_w = n_rows // NW
    groups = rows_per_w // GROUP
    stag_words = GROUP * ROW
    mesh = plsc.VectorSubcoreMesh(core_axis_name="c", subcore_axis_name="s")

    @functools.partial(
        pl.kernel,
        out_type=jax.ShapeDtypeStruct((n_rows * ROW,), jnp.int32),
        mesh=mesh,
        scratch_types=[
            pltpu.VMEM((stag_words,), jnp.float32),
            pltpu.VMEM((stag_words,), jnp.int32),
            pltpu.VMEM((TW,), jnp.int32),
            pltpu.VMEM((TW,), jnp.int32),
            pltpu.VMEM((TW,), jnp.int32),
            pltpu.VMEM((TW,), jnp.int32),
            pltpu.VMEM((4 * 256 * GROUP,), jnp.int32),
        ],
        compiler_params=pltpu.CompilerParams(needs_layout_passes=False),
    )
    def k(x_hbm, out_hbm, stag_in, stag_out, ka, kb, p1, p2, hist):
        c = lax.axis_index("c")
        s = lax.axis_index("s")
        wid = s * 2 + c
        iota = lax.iota(jnp.int32, 16)
        i17 = iota * 17
        zeros = jnp.zeros((16,), jnp.int32)
        ones = jnp.ones((16,), jnp.int32)
        low8 = iota < 8

        def transform(bits):
            sgn = lax.shift_right_arithmetic(bits, 31)
            m = bits ^ (sgn | INT_MIN)
            return ~m

        def do_group(g, _):
            row0 = wid * rows_per_w + g * GROUP
            base = row0 * ROW
            pltpu.sync_copy(x_hbm.at[pl.ds(base, stag_words)], stag_in)

            def zero_body(b, _):
                hist[pl.ds(b * 16, 16)] = zeros
                return 0

            lax.fori_loop(0, 4 * 256, zero_body, 0)

            def tin_row(r, _):
                def tin_body(j, _):
                    src = r * ROW + j * 16 + iota
                    src = jnp.minimum(src, stag_words - 1)
                    bits = plsc.bitcast(
                        plsc.load_gather(stag_in, [src]), jnp.int32)
                    key = transform(bits)
                    key = jnp.where((j * 16 + iota) < ROW, key, jnp.int32(-1))
                    dst = (j * 16 + iota) * 17 + r
                    plsc.store_scatter(ka, [dst], key)
                    for p in range(4):
                        d = lax.shift_right_logical(key, 8 * p) & 255
                        hidx = (d << 4) + (p * 4096 + r)
                        plsc.addupdate_scatter(hist, [hidx], ones)
                    return 0

                lax.fori_loop(0, 63, tin_body, 0)
                return 0

            lax.fori_loop(0, 16, tin_row, 0)

            for p in range(4):
                hoff = p * 4096

                def scan_body(b, run):
                    v = hist[pl.ds(hoff + b * 16, 16)]
                    hist[pl.ds(hoff + b * 16, 16)] = run
                    return run + v

                lax.fori_loop(0, 256, scan_body, zeros)

            def permute(src_k, src_p, dst_k, dst_p, p):
                hbase = p * 4096 + iota
                shift = 8 * p

                def body(j, _):
                    jvec = j * 17 + iota
                    key = plsc.load_gather(src_k, [jvec])
                    if src_p is None:
                        pay = jnp.full((16,), j, jnp.int32)
                    else:
                        pay = plsc.load_gather(src_p, [jvec])
                    d = lax.shift_right_logical(key, shift) & 255
                    hidx = (d << 4) + hbase
                    pos = plsc.load_gather(hist, [hidx])
                    dvec = pos * 17 + iota
                    if dst_k is not None:
                        plsc.store_scatter(dst_k, [dvec], key)
                    plsc.store_scatter(dst_p, [dvec], pay)
                    plsc.addupdate_scatter(hist, [hidx], ones)
                    return 0

                lax.fori_loop(0, PPAD, body, 0)

            permute(ka, None, kb, p1, 0)
            permute(kb, p1, ka, p2, 1)
            permute(ka, p2, kb, p1, 2)
            permute(kb, p1, None, p2, 3)

            def tout_row(r, _):
                def tout_body(j, _):
                    src = (j * 16 + iota) * 17 + r
                    v = plsc.load_gather(p2, [src])
                    dst = r * ROW + j * 16 + iota
                    dst = jnp.minimum(dst, stag_words - 1)
                    msk = (j * 16 + iota) < ROW
                    plsc.store_scatter(stag_out, [dst], v, mask=msk)
                    return 0

                lax.fori_loop(0, 63, tout_body, 0)
                return 0

            lax.fori_loop(0, 16, tout_row, 0)

            pltpu.sync_copy(stag_out, out_hbm.at[pl.ds(base, stag_words)])
            return 0

        lax.fori_loop(0, groups, do_group, 0)

    return k


@jax.jit
def kernel(inputs):
    n_rows = inputs.size // ROW
    out = _make(n_rows)(inputs)
    return out.reshape(n_rows, ROW)


# speedup vs baseline: 6.6748x; 1.8319x over previous
import functools

import jax
import jax.numpy as jnp
from jax import lax
from jax.experimental import pallas as pl
from jax.experimental.pallas import tpu as pltpu
from jax.experimental.pallas import tpu_sc as plsc

ROW = 1000
PPAD = 1008
GROUP = 16
NW = 32
T17 = PPAD * 17
T16 = PPAD * 16
INT_MIN = jnp.int32(-2147483648)
HI16 = jnp.int32(-65536)


def _make(n_rows):
    rows_per_w = n_rows // NW
    groups = rows_per_w // GROUP
    stag_words = GROUP * ROW
    mesh = plsc.VectorSubcoreMesh(core_axis_name="c", subcore_axis_name="s")

    @functools.partial(
        pl.kernel,
        out_type=jax.ShapeDtypeStruct((n_rows * ROW,), jnp.int32),
        mesh=mesh,
        scratch_types=[
            pltpu.VMEM((stag_words,), jnp.float32),
            pltpu.VMEM((stag_words,), jnp.int32),
            pltpu.VMEM((T17,), jnp.int32),
            pltpu.VMEM((T16,), jnp.int32),
            pltpu.VMEM((T16,), jnp.int32),
            pltpu.VMEM((4 * 256 * GROUP,), jnp.int32),
            pltpu.VMEM((256 * GROUP,), jnp.int32),
        ],
        compiler_params=pltpu.CompilerParams(needs_layout_passes=False),
    )
    def k(x_hbm, out_hbm, stag_in, stag_out, t17, t16a, t16b, hist, cur):
        c = lax.axis_index("c")
        s = lax.axis_index("s")
        wid = s * 2 + c
        iota = lax.iota(jnp.int32, 16)
        zeros = jnp.zeros((16,), jnp.int32)
        ones = jnp.ones((16,), jnp.int32)

        def transform(bits):
            sgn = lax.shift_right_arithmetic(bits, 31)
            m = bits ^ (sgn | INT_MIN)
            return ~m

        def do_group(g, _):
            row0 = wid * rows_per_w + g * GROUP
            base = row0 * ROW
            pltpu.sync_copy(x_hbm.at[pl.ds(base, stag_words)], stag_in)

            def zero_body(b, _):
                for m in range(4):
                    hist[pl.ds((b * 4 + m) * 16, 16)] = zeros
                return 0

            lax.fori_loop(0, 256, zero_body, 0)

            def tin_row(r, _):
                def tin_body(j, _):
                    src = r * ROW + j * 16 + iota
                    src = jnp.minimum(src, stag_words - 1)
                    bits = plsc.bitcast(
                        plsc.load_gather(stag_in, [src]), jnp.int32)
                    key = transform(bits)
                    key = jnp.where((j * 16 + iota) < ROW, key, jnp.int32(-1))
                    dst = (j * 16 + iota) * 17 + r
                    plsc.store_scatter(t17, [dst], key)
                    for p in range(4):
                        d = lax.shift_right_logical(key, 8 * p) & 255
                        hidx = (d << 4) + (p * 4096 + r)
                        plsc.addupdate_scatter(hist, [hidx], ones)
                    return 0

                lax.fori_loop(0, 63, tin_body, 0)
                return 0

            lax.fori_loop(0, 16, tin_row, 0)

            def make_scan(p):
                hoff = p * 4096

                def scan_body(blk, run):
                    for m in range(4):
                        b = blk * 4 + m
                        v = hist[pl.ds(hoff + b * 16, 16)]
                        hist[pl.ds(hoff + b * 16, 16)] = zeros
                        cur[pl.ds(b * 16, 16)] = run
                        run = run + v
                    return run

                lax.fori_loop(0, 64, scan_body, zeros)

            def permute(src, src_pitch, dst, dst_pitch, kind, shift):
                def body(blk, _):
                    j0 = blk * 4
                    keys, pays, hidxs, gs = [], [], [], []
                    for m in range(4):
                        jv = (j0 + m) * src_pitch + iota
                        key = plsc.load_gather(src, [jv])
                        keys.append(key)
                        if kind == 0:
                            pays.append(None)
                        elif kind == 1:
                            jv16 = (j0 + m) * 16 + iota
                            pays.append(plsc.load_gather(t16b, [jv16]))
                        else:
                            pays.append(key & jnp.int32(0xFFFF))
                        d = lax.shift_right_logical(key, shift) & 255
                        hidx = (d << 4) + iota
                        hidxs.append(hidx)
                    for m in range(4):
                        gs.append(plsc.load_gather(cur, [hidxs[m]]))
                    ds_ = [lax.shift_right_logical(k_, shift) & 255
                           for k_ in keys]
                    poss = []
                    for m in range(4):
                        pos = gs[m]
                        for mm in range(m):
                            pos = pos + jnp.where(ds_[m] == ds_[mm],
                                                  ones, zeros)
                        poss.append(pos)
                    for m in range(4):
                        dv = poss[m] * dst_pitch + iota
                        if kind == 0:
                            plsc.store_scatter(dst, [dv], keys[m])
                            plsc.store_scatter(
                                t16b, [dv],
                                jnp.full((16,), j0 + m, jnp.int32))
                        elif kind == 1:
                            packed = (keys[m] & HI16) | pays[m]
                            plsc.store_scatter(dst, [dv], packed)
                        elif kind == 2:
                            plsc.store_scatter(dst, [dv], keys[m])
                        else:
                            plsc.store_scatter(dst, [dv], pays[m])
                        plsc.addupdate_scatter(cur, [hidxs[m]], ones)
                    return 0

                lax.fori_loop(0, PPAD // 4, body, 0)

            make_scan(0)
            permute(t17, 17, t16a, 16, 0, 0)
            make_scan(1)
            permute(t16a, 16, t17, 16, 1, 8)
            make_scan(2)
            permute(t17, 16, t16a, 16, 2, 16)
            make_scan(3)
            permute(t16a, 16, t17, 17, 3, 24)

            def tout_row(r, _):
                def tout_body(j, _):
                    src = (j * 16 + iota) * 17 + r
                    v = plsc.load_gather(t17, [src])
                    dst = r * ROW + j * 16 + iota
                    dst = jnp.minimum(dst, stag_words - 1)
                    msk = (j * 16 + iota) < ROW
                    plsc.store_scatter(stag_out, [dst], v, mask=msk)
                    return 0

                lax.fori_loop(0, 63, tout_body, 0)
                return 0

            lax.fori_loop(0, 16, tout_row, 0)

            pltpu.sync_copy(stag_out, out_hbm.at[pl.ds(base, stag_words)])
            return 0

        lax.fori_loop(0, groups, do_group, 0)

    return k


@jax.jit
def kernel(inputs):
    n_rows = inputs.size // ROW
    out = _make(n_rows)(inputs)
    return out.reshape(n_rows, ROW)


# speedup vs baseline: 8.0271x; 1.2026x over previous
import functools

import jax
import jax.numpy as jnp
from jax import lax
from jax.experimental import pallas as pl
from jax.experimental.pallas import tpu as pltpu
from jax.experimental.pallas import tpu_sc as plsc

ROW = 1000
PPAD = 1008
GROUP = 16
NW = 32
T17 = PPAD * 17
T16 = PPAD * 16
INT_MIN = jnp.int32(-2147483648)
HI16 = jnp.int32(-65536)


def _make(n_rows):
    rows_per_w = n_rows // NW
    groups = rows_per_w // GROUP
    stag_words = GROUP * ROW
    mesh = plsc.VectorSubcoreMesh(core_axis_name="c", subcore_axis_name="s")

    @functools.partial(
        pl.kernel,
        out_type=jax.ShapeDtypeStruct((n_rows * ROW,), jnp.int32),
        mesh=mesh,
        scratch_types=[
            pltpu.VMEM((stag_words,), jnp.float32),
            pltpu.VMEM((stag_words,), jnp.int32),
            pltpu.VMEM((T17,), jnp.int32),
            pltpu.VMEM((T16,), jnp.int32),
            pltpu.VMEM((T16,), jnp.int32),
            pltpu.VMEM((4 * 256 * GROUP,), jnp.int32),
            pltpu.VMEM((256 * GROUP,), jnp.int32),
            pltpu.SemaphoreType.DMA,
            pltpu.SemaphoreType.DMA,
        ],
        compiler_params=pltpu.CompilerParams(needs_layout_passes=False),
    )
    def k(x_hbm, out_hbm, stag_in, stag_out, t17, t16a, t16b, hist, cur,
          sem_in, sem_out):
        c = lax.axis_index("c")
        s = lax.axis_index("s")
        wid = s * 2 + c
        iota = lax.iota(jnp.int32, 16)
        zeros = jnp.zeros((16,), jnp.int32)
        ones = jnp.ones((16,), jnp.int32)

        def transform(bits):
            sgn = lax.shift_right_arithmetic(bits, 31)
            m = bits ^ (sgn | INT_MIN)
            return ~m

        def zero_body(b, _):
            for m in range(4):
                hist[pl.ds((b * 4 + m) * 16, 16)] = zeros
            return 0

        lax.fori_loop(0, 256, zero_body, 0)

        wbase = wid * rows_per_w * ROW
        pltpu.sync_copy(x_hbm.at[pl.ds(wbase, stag_words)], stag_in)

        def do_group(g, _):
            base = wbase + g * stag_words
            nbase = base + stag_words

            def tin_row(r, _):
                def tin_body(j, _):
                    src = r * ROW + j * 16 + iota
                    src = jnp.minimum(src, stag_words - 1)
                    bits = plsc.bitcast(
                        plsc.load_gather(stag_in, [src]), jnp.int32)
                    key = transform(bits)
                    key = jnp.where((j * 16 + iota) < ROW, key, jnp.int32(-1))
                    dst = (j * 16 + iota) * 17 + r
                    plsc.store_scatter(t17, [dst], key)
                    for p in range(4):
                        d = lax.shift_right_logical(key, 8 * p) & 255
                        hidx = (d << 4) + (p * 4096 + r)
                        plsc.addupdate_scatter(hist, [hidx], ones)
                    return 0

                lax.fori_loop(0, 63, tin_body, 0)
                return 0

            lax.fori_loop(0, 16, tin_row, 0)

            @pl.when(g + 1 < groups)
            def _():
                pltpu.make_async_copy(
                    x_hbm.at[pl.ds(nbase, stag_words)], stag_in,
                    sem_in).start()

            def make_scan(p):
                hoff = p * 4096

                def scan_body(blk, run):
                    for m in range(4):
                        b = blk * 4 + m
                        v = hist[pl.ds(hoff + b * 16, 16)]
                        hist[pl.ds(hoff + b * 16, 16)] = zeros
                        cur[pl.ds(b * 16, 16)] = run
                        run = run + v
                    return run

                lax.fori_loop(0, 64, scan_body, zeros)

            def permute(src, src_pitch, dst, dst_pitch, kind, shift):
                def load_block(j0):
                    keys = []
                    pays = []
                    for m in range(4):
                        jv = (j0 + m) * src_pitch + iota
                        keys.append(plsc.load_gather(src, [jv]))
                        if kind == 1:
                            jv16 = (j0 + m) * 16 + iota
                            pays.append(plsc.load_gather(t16b, [jv16]))
                    return tuple(keys) + tuple(pays)

                def proc_block(j0, blkdata):
                    keys = blkdata[:4]
                    ds_ = [lax.shift_right_logical(k_, shift) & 255
                           for k_ in keys]
                    hidxs = [(d << 4) + iota for d in ds_]
                    gs = [plsc.load_gather(cur, [hidxs[m]]) for m in range(4)]
                    for m in range(4):
                        pos = gs[m]
                        for mm in range(m):
                            pos = pos + jnp.where(ds_[m] == ds_[mm],
                                                  ones, zeros)
                        dv = pos * dst_pitch + iota
                        if kind == 0:
                            plsc.store_scatter(dst, [dv], keys[m])
                            plsc.store_scatter(
                                t16b, [dv],
                                jnp.full((16,), j0 + m, jnp.int32))
                        elif kind == 1:
                            packed = (keys[m] & HI16) | blkdata[4 + m]
                            plsc.store_scatter(dst, [dv], packed)
                        elif kind == 2:
                            plsc.store_scatter(dst, [dv], keys[m])
                        else:
                            plsc.store_scatter(dst, [dv],
                                               keys[m] & jnp.int32(0xFFFF))
                        plsc.addupdate_scatter(cur, [hidxs[m]], ones)

                def body(blk, carry):
                    nxt = load_block((blk + 1) * 4)
                    proc_block(blk * 4, carry)
                    return nxt

                nblk = PPAD // 4
                last = lax.fori_loop(0, nblk - 1, body, load_block(0))
                proc_block((nblk - 1) * 4, last)

            make_scan(0)
            permute(t17, 17, t16a, 16, 0, 0)
            make_scan(1)
            permute(t16a, 16, t17, 16, 1, 8)
            make_scan(2)
            permute(t17, 16, t16a, 16, 2, 16)
            make_scan(3)
            permute(t16a, 16, t17, 17, 3, 24)

            @pl.when(g > 0)
            def _():
                pltpu.make_async_copy(
                    stag_out, out_hbm.at[pl.ds(base - stag_words, stag_words)],
                    sem_out).wait()

            def tout_row(r, _):
                def tout_body(j, _):
                    src = (j * 16 + iota) * 17 + r
                    v = plsc.load_gather(t17, [src])
                    dst = r * ROW + j * 16 + iota
                    dst = jnp.minimum(dst, stag_words - 1)
                    msk = (j * 16 + iota) < ROW
                    plsc.store_scatter(stag_out, [dst], v, mask=msk)
                    return 0

                lax.fori_loop(0, 63, tout_body, 0)
                return 0

            lax.fori_loop(0, 16, tout_row, 0)

            pltpu.make_async_copy(
                stag_out, out_hbm.at[pl.ds(base, stag_words)],
                sem_out).start()

            @pl.when(g + 1 < groups)
            def _():
                pltpu.make_async_copy(
                    x_hbm.at[pl.ds(nbase, stag_words)], stag_in,
                    sem_in).wait()

            return 0

        lax.fori_loop(0, groups, do_group, 0)
        pltpu.make_async_copy(
            stag_out,
            out_hbm.at[pl.ds(wbase + (groups - 1) * stag_words, stag_words)],
            sem_out).wait()

    return k


@jax.jit
def kernel(inputs):
    n_rows = inputs.size // ROW
    out = _make(n_rows)(inputs)
    return out.reshape(n_rows, ROW)


# speedup vs baseline: 8.5985x; 1.0712x over previous
import functools

import jax
import jax.numpy as jnp
from jax import lax
from jax.experimental import pallas as pl
from jax.experimental.pallas import tpu as pltpu
from jax.experimental.pallas import tpu_sc as plsc

ROW = 1000
PPAD = 1008
GROUP = 16
NW = 32
T17 = PPAD * 17
T16 = PPAD * 16
INT_MIN = jnp.int32(-2147483648)
LOW10 = jnp.int32(1023)
NLOW10 = jnp.int32(-1024)
DIGITS = [(0, 1023), (10, 255), (18, 127), (25, 127)]
HB_OFF = [0, 0, 4096, 6144]


def _make(n_rows):
    rows_per_w = n_rows // NW
    groups = rows_per_w // GROUP
    stag_words = GROUP * ROW
    mesh = plsc.VectorSubcoreMesh(core_axis_name="c", subcore_axis_name="s")

    @functools.partial(
        pl.kernel,
        out_type=jax.ShapeDtypeStruct((n_rows * ROW,), jnp.int32),
        mesh=mesh,
        scratch_types=[
            pltpu.VMEM((stag_words,), jnp.float32),
            pltpu.VMEM((stag_words,), jnp.int32),
            pltpu.VMEM((T17,), jnp.int32),
            pltpu.VMEM((T16,), jnp.int32),
            pltpu.VMEM((1024 * GROUP,), jnp.int32),
            pltpu.VMEM((512 * GROUP,), jnp.int32),
            pltpu.SemaphoreType.DMA,
            pltpu.SemaphoreType.DMA,
        ],
        compiler_params=pltpu.CompilerParams(needs_layout_passes=False),
    )
    def k(x_hbm, out_hbm, stag_in, stag_out, t17, t16a, hista, histb,
          sem_in, sem_out):
        c = lax.axis_index("c")
        s = lax.axis_index("s")
        wid = s * 2 + c
        iota = lax.iota(jnp.int32, 16)
        zeros = jnp.zeros((16,), jnp.int32)
        ones = jnp.ones((16,), jnp.int32)

        def transform(bits):
            sgn = lax.shift_right_arithmetic(bits, 31)
            m = bits ^ (sgn | INT_MIN)
            return ~m

        def zero_loop(ref, nvregs):
            def body(b, _):
                for m in range(4):
                    ref[pl.ds((b * 4 + m) * 16, 16)] = zeros
                return 0

            lax.fori_loop(0, nvregs // 4, body, 0)

        def scan(ref, off, nbins):
            def body(blk, run):
                for m in range(4):
                    b = off + (blk * 4 + m) * 16
                    v = ref[pl.ds(b, 16)]
                    ref[pl.ds(b, 16)] = run
                    run = run + v
                return run

            lax.fori_loop(0, nbins // 4, body, zeros)

        zero_loop(hista, 1024)
        zero_loop(histb, 512)

        wbase = wid * rows_per_w * ROW
        pltpu.sync_copy(x_hbm.at[pl.ds(wbase, stag_words)], stag_in)

        def do_group(g, _):
            base = wbase + g * stag_words
            nbase = base + stag_words

            def tin_row(r, _):
                def tin_body(j, _):
                    src = r * ROW + j * 16 + iota
                    src = jnp.minimum(src, stag_words - 1)
                    bits = plsc.bitcast(
                        plsc.load_gather(stag_in, [src]), jnp.int32)
                    key = transform(bits)
                    key = jnp.where((j * 16 + iota) < ROW, key, jnp.int32(-1))
                    dst = (j * 16 + iota) * 17 + r
                    plsc.store_scatter(t17, [dst], key)
                    return 0

                lax.fori_loop(0, 63, tin_body, 0)
                return 0

            lax.fori_loop(0, 16, tin_row, 0)

            @pl.when(g + 1 < groups)
            def _():
                pltpu.make_async_copy(
                    x_hbm.at[pl.ds(nbase, stag_words)], stag_in,
                    sem_in).start()

            def ha_body(j, _):
                key = plsc.load_gather(t17, [j * 17 + iota])
                hidx = ((key & LOW10) << 4) + iota
                plsc.addupdate_scatter(hista, [hidx], ones)
                return 0

            lax.fori_loop(0, PPAD, ha_body, 0)
            scan(hista, 0, 1024)

            def permute(src, src_pitch, dst, dst_pitch, cur, hoff, p, kind):
                shift, mask = DIGITS[p]
                hvec = iota + hoff

                def load_block(j0):
                    return tuple(
                        plsc.load_gather(src, [(j0 + m) * src_pitch + iota])
                        for m in range(4))

                def proc_block(j0, keys):
                    ds_ = [lax.shift_right_logical(k_, shift) & mask
                           for k_ in keys]
                    hidxs = [(d << 4) + hvec for d in ds_]
                    gs = [plsc.load_gather(cur, [hidxs[m]]) for m in range(4)]
                    for m in range(4):
                        pos = gs[m]
                        for mm in range(m):
                            pos = pos + jnp.where(ds_[m] == ds_[mm],
                                                  ones, zeros)
                        dv = pos * dst_pitch + iota
                        if kind == 0:
                            plsc.store_scatter(
                                dst, [dv], (keys[m] & NLOW10) | (j0 + m))
                        elif kind == 1:
                            plsc.store_scatter(dst, [dv], keys[m])
                        else:
                            plsc.store_scatter(dst, [dv], keys[m] & LOW10)
                        plsc.addupdate_scatter(cur, [hidxs[m]], ones)

                def body(blk, carry):
                    nxt = load_block((blk + 1) * 4)
                    proc_block(blk * 4, carry)
                    return nxt

                nblk = PPAD // 4
                last = lax.fori_loop(0, nblk - 1, body, load_block(0))
                proc_block((nblk - 1) * 4, last)

            permute(t17, 17, t16a, 16, hista, 0, 0, 0)

            def hb_body(j, _):
                key = plsc.load_gather(t16a, [j * 16 + iota])
                for p in (1, 2, 3):
                    shift, mask = DIGITS[p]
                    d = lax.shift_right_logical(key, shift) & mask
                    hidx = (d << 4) + iota + HB_OFF[p]
                    plsc.addupdate_scatter(histb, [hidx], ones)
                return 0

            lax.fori_loop(0, PPAD, hb_body, 0)

            scan(histb, 0, 256)
            permute(t16a, 16, t17, 16, histb, HB_OFF[1], 1, 1)
            scan(histb, 4096, 128)
            permute(t17, 16, t16a, 16, histb, HB_OFF[2], 2, 1)
            scan(histb, 6144, 128)
            permute(t16a, 16, t17, 17, histb, HB_OFF[3], 3, 2)

            @pl.when(g > 0)
            def _():
                pltpu.make_async_copy(
                    stag_out, out_hbm.at[pl.ds(base - stag_words, stag_words)],
                    sem_out).wait()

            def tout_row(r, _):
                def tout_body(j, _):
                    src = (j * 16 + iota) * 17 + r
                    v = plsc.load_gather(t17, [src])
                    dst = r * ROW + j * 16 + iota
                    dst = jnp.minimum(dst, stag_words - 1)
                    msk = (j * 16 + iota) < ROW
                    plsc.store_scatter(stag_out, [dst], v, mask=msk)
                    hista[pl.ds((r * 63 + j) * 16, 16)] = zeros
                    return 0

                lax.fori_loop(0, 63, tout_body, 0)
                return 0

            lax.fori_loop(0, 16, tout_row, 0)
            for b in range(1008, 1024):
                hista[pl.ds(b * 16, 16)] = zeros
            zero_loop(histb, 512)

            pltpu.make_async_copy(
                stag_out, out_hbm.at[pl.ds(base, stag_words)],
                sem_out).start()

            @pl.when(g + 1 < groups)
            def _():
                pltpu.make_async_copy(
                    x_hbm.at[pl.ds(nbase, stag_words)], stag_in,
                    sem_in).wait()

            return 0

        lax.fori_loop(0, groups, do_group, 0)
        pltpu.make_async_copy(
            stag_out,
            out_hbm.at[pl.ds(wbase + (groups - 1) * stag_words, stag_words)],
            sem_out).wait()

    return k


@jax.jit
def kernel(inputs):
    n_rows = inputs.size // ROW
    out = _make(n_rows)(inputs)
    return out.reshape(n_rows, ROW)
